# Initial kernel scaffold; baseline (speedup 1.0000x reference)
import jax
import jax.numpy as jnp
from jax.experimental import pallas as pl


def kernel(x, edge_index, ppi_list, idx, w1a, b1a, w1b, b1b, eps0, bn0_g, bn0_b, w2a, b2a, w2b, b2b, eps1, bn1_g, bn1_b, lin_w, lin_b, int_wa, int_ba, int_wb, int_bb):
    raise NotImplementedError("write your pallas kernel here")


# speedup vs baseline: 4.2530x; 4.2530x over previous
import functools

import jax
import jax.numpy as jnp
from jax import lax
from jax.experimental import pallas as pl
from jax.experimental.pallas import tpu as pltpu
from jax.experimental.pallas import tpu_sc as plsc

N = 10000
E = 320000
D = 128
H = 128
OUT = 7
P = 100000

NC = 2
NS = 16
NW = NC * NS
HD = D // NC

EPT = E // NS
SEG_CH = 100
SEG_NCH = EPT // SEG_CH

P_PAD = 102400
PPW = P_PAD // NW
PAIR_CH = 128
PAIR_NCH = PPW // PAIR_CH

ZR = 624
ZR_LAST = N - (NS - 1) * ZR


@functools.cache
def _sc_mesh():
    return plsc.VectorSubcoreMesh(core_axis_name="c", subcore_axis_name="s",
                                  num_cores=NC, num_subcores=NS)


def _stripe_copy(src_ref, dst_ref, s):
    off = pl.multiple_of(s * ZR, 8)

    @pl.when(s < NS - 1)
    def _():
        pltpu.sync_copy(src_ref.at[pl.ds(off, ZR)],
                        dst_ref.at[pl.ds(off, ZR)])

    @pl.when(s == NS - 1)
    def _():
        pltpu.sync_copy(src_ref.at[pl.ds((NS - 1) * ZR, ZR_LAST)],
                        dst_ref.at[pl.ds((NS - 1) * ZR, ZR_LAST)])


def _segsum_body(hs_hbm, src_hbm, dst_hbm, zero_hbm, out_hbm,
                 src_v, dst_v, bufa, bufb, acc, sema, semb):
    c = lax.axis_index("c")
    s = lax.axis_index("s")

    _stripe_copy(zero_hbm, acc, s)
    pltpu.sync_copy(src_hbm.at[s], src_v)
    pltpu.sync_copy(dst_hbm.at[s], dst_v)
    plsc.subcore_barrier()

    h_hbm = hs_hbm.at[c]

    def body(i, carry):
        j0 = 2 * i
        j1 = j0 + 1
        ca = pltpu.async_copy(h_hbm.at[src_v.at[j0]], bufa, sema)
        cb = pltpu.async_copy(h_hbm.at[src_v.at[j1]], bufb, semb)
        ca.wait()
        pltpu.sync_copy(bufa, acc.at[dst_v.at[j0]], add=True)
        cb.wait()
        pltpu.sync_copy(bufb, acc.at[dst_v.at[j1]], add=True)
        return carry

    lax.fori_loop(0, SEG_NCH // 2, body, 0)
    plsc.subcore_barrier()
    _stripe_copy(acc, out_hbm.at[c], s)


def _segsum(hs, src, dst, zero):
    k = pl.kernel(
        _segsum_body,
        out_type=jax.ShapeDtypeStruct((NC, N, HD), jnp.float32),
        mesh=_sc_mesh(),
        compiler_params=pltpu.CompilerParams(use_tc_tiling_on_sc=False),
        scratch_types=[
            pltpu.VMEM((SEG_NCH, SEG_CH), jnp.int32),
            pltpu.VMEM((SEG_NCH, SEG_CH), jnp.int32),
            pltpu.VMEM((SEG_CH, HD), jnp.float32),
            pltpu.VMEM((SEG_CH, HD), jnp.float32),
            pltpu.VMEM_SHARED((N, HD), jnp.float32),
            pltpu.SemaphoreType.DMA,
            pltpu.SemaphoreType.DMA,
        ],
    )
    return k(hs, src, dst, zero)


def _pair_body(u_hbm, v_hbm, p1_hbm, p2_hbm, g_hbm,
               p1_v, p2_v, bufa, bufb, sema, semb):
    c = lax.axis_index("c")
    s = lax.axis_index("s")
    wid = c * NS + s
    base = wid * PPW

    pltpu.sync_copy(p1_hbm.at[wid], p1_v)
    pltpu.sync_copy(p2_hbm.at[wid], p2_v)

    def body(j, carry):
        ca = pltpu.async_copy(u_hbm.at[p1_v.at[j]], bufa, sema)
        cb = pltpu.async_copy(v_hbm.at[p2_v.at[j]], bufb, semb)
        ca.wait()
        cb.wait()

        def radd(r, cr):
            for k in range(D // 16):
                sl = pl.ds(k * 16, 16)
                bufa[r, sl] = bufa[r, sl] + bufb[r, sl]
            return cr

        lax.fori_loop(0, PAIR_CH, radd, 0)
        pltpu.sync_copy(bufa, g_hbm.at[pl.ds(base + j * PAIR_CH, PAIR_CH)])
        return carry

    lax.fori_loop(0, PAIR_NCH, body, 0)


def _pair(u, v, p1, p2):
    k = pl.kernel(
        _pair_body,
        out_type=jax.ShapeDtypeStruct((P_PAD, D), jnp.float32),
        mesh=_sc_mesh(),
        scratch_types=[
            pltpu.VMEM((PAIR_NCH, PAIR_CH), jnp.int32),
            pltpu.VMEM((PAIR_NCH, PAIR_CH), jnp.int32),
            pltpu.VMEM((PAIR_CH, D), jnp.float32),
            pltpu.VMEM((PAIR_CH, D), jnp.float32),
            pltpu.SemaphoreType.DMA,
            pltpu.SemaphoreType.DMA,
        ],
    )
    return k(u, v, p1, p2)


def _gin_tc_body(eps_ref, h_ref, agg_ref, wa_ref, ba_ref, wb_ref, bb_ref,
                 g_ref, b_ref, out_ref):
    h = jnp.concatenate([h_ref[0], h_ref[1]], axis=1)
    agg = jnp.concatenate([agg_ref[0], agg_ref[1]], axis=1)
    t = h * (1.0 + eps_ref[0]) + agg
    t = jnp.dot(t, wa_ref[...], preferred_element_type=jnp.float32)
    t = jnp.maximum(t + ba_ref[...], 0.0)
    t = jnp.dot(t, wb_ref[...], preferred_element_type=jnp.float32)
    t = jnp.maximum(t + bb_ref[...], 0.0)
    m = jnp.mean(t, axis=0, keepdims=True)
    var = jnp.mean(t * t, axis=0, keepdims=True) - m * m
    t = (t - m) * lax.rsqrt(var + 1e-5) * g_ref[...] + b_ref[...]
    t = jnp.maximum(t, 0.0)
    out_ref[0] = t[:, :HD]
    out_ref[1] = t[:, HD:]


def _gin_tc(hs, aggs, wa, ba, wb, bb, eps, bng, bnb):
    vm = pl.BlockSpec(memory_space=pltpu.VMEM)
    return pl.pallas_call(
        _gin_tc_body,
        out_shape=jax.ShapeDtypeStruct((NC, N, HD), jnp.float32),
        in_specs=[pl.BlockSpec(memory_space=pltpu.SMEM)] + [vm] * 8,
        out_specs=vm,
    )(eps.reshape(1), hs, aggs, wa, ba.reshape(1, H), wb, bb.reshape(1, H),
      bng.reshape(1, H), bnb.reshape(1, H))


def _final_tc_body(h_ref, lw_ref, lb_ref, wat_ref, wab_ref, iba_ref,
                   u_ref, v_ref):
    h2 = jnp.concatenate([h_ref[0], h_ref[1]], axis=1)
    h = jnp.dot(h2, lw_ref[...], preferred_element_type=jnp.float32)
    h = jnp.maximum(h + lb_ref[...], 0.0)
    u_ref[...] = jnp.dot(h, wat_ref[...],
                         preferred_element_type=jnp.float32) + iba_ref[...]
    v_ref[...] = jnp.dot(h, wab_ref[...], preferred_element_type=jnp.float32)


def _final_tc(hs2, lin_w, lin_b, wat, wab, iba):
    return pl.pallas_call(
        _final_tc_body,
        out_shape=(jax.ShapeDtypeStruct((N, H), jnp.float32),
                   jax.ShapeDtypeStruct((N, H), jnp.float32)),
    )(hs2, lin_w, lin_b.reshape(1, H), wat, wab, iba.reshape(1, H))


_HEAD_BLK = 2048


def _head_tc_body(g_ref, wb_ref, bb_ref, out_ref):
    z = jnp.maximum(g_ref[...], 0.0)
    out_ref[...] = jnp.dot(z, wb_ref[...],
                           preferred_element_type=jnp.float32) + bb_ref[...]


def _head_tc(g, int_wb, int_bb):
    nblk = (P + _HEAD_BLK - 1) // _HEAD_BLK
    return pl.pallas_call(
        _head_tc_body,
        grid=(nblk,),
        in_specs=[
            pl.BlockSpec((_HEAD_BLK, H), lambda i: (i, 0)),
            pl.BlockSpec((H, OUT), lambda i: (0, 0)),
            pl.BlockSpec((1, OUT), lambda i: (0, 0)),
        ],
        out_specs=pl.BlockSpec((_HEAD_BLK, OUT), lambda i: (i, 0)),
        out_shape=jax.ShapeDtypeStruct((P, OUT), jnp.float32),
    )(g, int_wb, int_bb.reshape(1, OUT))


def kernel(x, edge_index, ppi_list, idx,
           w1a, b1a, w1b, b1b, eps0, bn0_g, bn0_b,
           w2a, b2a, w2b, b2b, eps1, bn1_g, bn1_b,
           lin_w, lin_b, int_wa, int_ba, int_wb, int_bb):
    src = edge_index[0].reshape(NS, SEG_NCH, SEG_CH)
    dst = edge_index[1].reshape(NS, SEG_NCH, SEG_CH)
    zero = jnp.zeros((N, HD), jnp.float32)

    xs = jnp.stack([x[:, :HD], x[:, HD:]])
    agg0 = _segsum(xs, src, dst, zero)
    h1 = _gin_tc(xs, agg0, w1a, b1a, w1b, b1b, eps0, bn0_g, bn0_b)
    agg1 = _segsum(h1, src, dst, zero)
    h2 = _gin_tc(h1, agg1, w2a, b2a, w2b, b2b, eps1, bn1_g, bn1_b)

    u, v = _final_tc(h2, lin_w, lin_b, int_wa[:H], int_wa[H:], int_ba)

    pairs = jnp.take(ppi_list, idx, axis=0)
    pad = jnp.zeros((P_PAD - P,), jnp.int32)
    p1 = jnp.concatenate([pairs[:, 0], pad]).reshape(NW, PAIR_NCH, PAIR_CH)
    p2 = jnp.concatenate([pairs[:, 1], pad]).reshape(NW, PAIR_NCH, PAIR_CH)

    g = _pair(u, v, p1, p2)
    return _head_tc(g, int_wb, int_bb)
